# Initial kernel scaffold; baseline (speedup 1.0000x reference)
import jax
import jax.numpy as jnp
from jax.experimental import pallas as pl


def kernel(x, init_state):
    raise NotImplementedError("write your pallas kernel here")


# speedup vs baseline: 2.8087x; 2.8087x over previous
import numpy as np
import jax
import jax.numpy as jnp
from jax.experimental import pallas as pl
from jax.experimental.pallas import tpu as pltpu

_ALPHA = 0.99
_EPS = 1e-14
_L = 200


def _decay_matrix(L: int) -> np.ndarray:
    t = np.arange(L, dtype=np.float64)
    W = np.where(
        t[:, None] >= t[None, :],
        (1.0 - _ALPHA) * _ALPHA ** (t[:, None] - t[None, :]),
        0.0,
    )
    aug = np.zeros((L, L + 8), dtype=np.float64)
    aug[:, 0] = _ALPHA ** (t + 1.0)
    aug[:, 8:] = W
    return aug.astype(np.float32)


def _eun_kernel(x_ref, w_ref, s0_ref, out_ref, carry_ref):
    j = pl.program_id(1)

    @pl.when(j == 0)
    def _init():
        carry_ref[...] = jnp.zeros_like(carry_ref)
        carry_ref[0:1, :] = s0_ref[...]

    val = x_ref[0]
    x2 = val * val
    ps = x2 + pltpu.roll(x2, x2.shape[1] - 1, axis=1)
    m = jnp.sqrt(jnp.maximum(ps, _EPS))
    lane = jax.lax.broadcasted_iota(jnp.int32, m.shape, 1)
    even = (lane % 2) == 0
    m = jnp.where(even, m, 0.0)
    m_aug = jnp.concatenate([carry_ref[...], m], axis=0)
    states = jnp.dot(w_ref[...], m_aug, preferred_element_type=jnp.float32)
    carry_ref[0:1, :] = states[_L - 1 : _L, :]
    r = jax.lax.rsqrt(states)
    r_full = jnp.where(even, r, pltpu.roll(r, 1, axis=1))
    out_ref[0] = val * r_full


def kernel(x, init_state):
    b, c, t, f, _ = x.shape
    bc = b * c
    f2 = f * 2
    xr = x.reshape(bc, t, f2)
    s0 = jnp.repeat(init_state.reshape(f), 2).reshape(1, f2)
    w_aug = jnp.asarray(_decay_matrix(_L))
    n_chunks = t // _L

    out = pl.pallas_call(
        _eun_kernel,
        out_shape=jax.ShapeDtypeStruct((bc, t, f2), x.dtype),
        grid=(bc, n_chunks),
        in_specs=[
            pl.BlockSpec((1, _L, f2), lambda i, j: (i, j, 0)),
            pl.BlockSpec((_L, _L + 8), lambda i, j: (0, 0)),
            pl.BlockSpec((1, f2), lambda i, j: (0, 0)),
        ],
        out_specs=pl.BlockSpec((1, _L, f2), lambda i, j: (i, j, 0)),
        scratch_shapes=[pltpu.VMEM((8, f2), jnp.float32)],
        compiler_params=pltpu.CompilerParams(
            dimension_semantics=("parallel", "arbitrary"),
        ),
        name="exp_unit_norm",
    )(xr, w_aug, s0)

    return out.reshape(b, c, t, f, 2)


# speedup vs baseline: 14.0747x; 5.0110x over previous
import numpy as np
import jax
import jax.numpy as jnp
from jax.experimental import pallas as pl
from jax.experimental.pallas import tpu as pltpu

_ALPHA = 0.99
_EPS = 1e-14
_LC = 256


def _scan_mats(L: int):
    t = np.arange(L, dtype=np.float64)
    U = np.where(
        t[:, None] <= t[None, :],
        (1.0 - _ALPHA) * _ALPHA ** (t[None, :] - t[:, None]),
        0.0,
    )
    a_row = _ALPHA ** (t + 1.0)
    return U.astype(np.float32), a_row.reshape(1, L).astype(np.float32)


def _eun_kernel(x_ref, u_ref, a_ref, s0_ref, o_ref):
    val = x_ref[0]
    v2 = val * val
    mag2 = v2[:, 0, :] + v2[:, 1, :]
    m = jnp.sqrt(jnp.maximum(mag2, _EPS))
    carry = s0_ref[...]
    u_full = u_ref[...]
    a_full = a_ref[...]
    t = m.shape[1]
    o = 0
    while o < t:
        L = min(_LC, t - o)
        m_c = m[:, o : o + L]
        u = u_full[:L, :L]
        a = a_full[:, :L]
        states = (
            jnp.dot(m_c, u, preferred_element_type=jnp.float32) + carry * a
        )
        carry = states[:, L - 1 : L]
        r = jax.lax.rsqrt(states)
        r_pair = jnp.broadcast_to(r[:, None, :], (r.shape[0], 2, L))
        o_ref[0, :, :, o : o + L] = val[:, :, o : o + L] * r_pair
        o += L


def kernel(x, init_state):
    b, c, t, f, p = x.shape
    bc = b * c
    xt = jnp.transpose(x, (0, 1, 3, 4, 2)).reshape(bc, f, p, t)
    s0 = init_state.reshape(f, 1)
    u_np, a_np = _scan_mats(_LC)

    out = pl.pallas_call(
        _eun_kernel,
        out_shape=jax.ShapeDtypeStruct((bc, f, p, t), x.dtype),
        grid=(bc,),
        in_specs=[
            pl.BlockSpec((1, f, p, t), lambda i: (i, 0, 0, 0)),
            pl.BlockSpec((_LC, _LC), lambda i: (0, 0)),
            pl.BlockSpec((1, _LC), lambda i: (0, 0)),
            pl.BlockSpec((f, 1), lambda i: (0, 0)),
        ],
        out_specs=pl.BlockSpec((1, f, p, t), lambda i: (i, 0, 0, 0)),
        compiler_params=pltpu.CompilerParams(
            dimension_semantics=("arbitrary",),
        ),
        name="exp_unit_norm",
    )(xt, jnp.asarray(u_np), jnp.asarray(a_np), s0)

    return jnp.transpose(out.reshape(b, c, f, p, t), (0, 1, 4, 2, 3))
